# baseline (speedup 1.0000x reference)
import functools

import jax
import jax.numpy as jnp
from jax import lax
from jax.experimental import pallas as pl
from jax.experimental.pallas import tpu as pltpu
from jax.experimental.pallas import tpu_sc as plsc

SEQ, HID = 2048, 1024
NH, HD = 16, 64
E, TOPK, DFF = 8, 2, 2048
QKV_OUT = (NH + 2 * NH) * HD

ST = 256
QT = 256
ET = 128
NPAIR = SEQ * TOPK
P = 5120
NT = P // ET
NTE = 48

NC, NS, L = 2, 16, 16
NW = NC * NS
TPW = SEQ // NW


def _k1_body(x_ref, g_ref, b_ref, wq_ref, wk_ref, wv_ref,
             q_ref, k_ref, v_ref):
    x = x_ref[...]
    mu = jnp.mean(x, axis=-1, keepdims=True)
    var = jnp.mean((x - mu) ** 2, axis=-1, keepdims=True)
    ln = (x - mu) * lax.rsqrt(var + 1e-5) * g_ref[...] + b_ref[...]
    qa = jnp.dot(ln, wq_ref[...], preferred_element_type=jnp.float32)
    ka = jnp.dot(ln, wk_ref[...], preferred_element_type=jnp.float32)
    va = jnp.dot(ln, wv_ref[...], preferred_element_type=jnp.float32)
    for h in range(NH):
        q_ref[h] = qa[:, HD * h:HD * (h + 1)]
        k_ref[h] = ka[:, HD * h:HD * (h + 1)]
        v_ref[h] = va[:, HD * h:HD * (h + 1)]


def _k1(hs2d, ln1w, ln1b, wq, wk, wv):
    hspec = pl.BlockSpec((NH, ST, HD), lambda i: (0, i, 0))
    return pl.pallas_call(
        _k1_body,
        grid=(SEQ // ST,),
        in_specs=[
            pl.BlockSpec((ST, HID), lambda i: (i, 0)),
            pl.BlockSpec((1, HID), lambda i: (0, 0)),
            pl.BlockSpec((1, HID), lambda i: (0, 0)),
            pl.BlockSpec((HID, HID), lambda i: (0, 0)),
            pl.BlockSpec((HID, HID), lambda i: (0, 0)),
            pl.BlockSpec((HID, HID), lambda i: (0, 0)),
        ],
        out_specs=[hspec, hspec, hspec],
        out_shape=[jax.ShapeDtypeStruct((NH, SEQ, HD), jnp.float32)] * 3,
    )(hs2d, ln1w, ln1b, wq, wk, wv)


NQ = SEQ // QT
NV = NQ * (NQ + 1) // 2


def _attn_body(rmap_ref, cmap_ref, q_ref, k_ref, v_ref, o_ref, l_scr):
    i = pl.program_id(1)
    r = rmap_ref[i]
    c = cmap_ref[i]
    scale = 1.0 / (HD ** 0.5)
    q = q_ref[0] * scale
    kb = k_ref[0, pl.ds(c * QT, QT), :]
    vb = v_ref[0, pl.ds(c * QT, QT), :]
    s = lax.dot_general(q, kb, (((1,), (1,)), ((), ())),
                        preferred_element_type=jnp.float32)
    ri = lax.broadcasted_iota(jnp.int32, (QT, QT), 0)
    ci = lax.broadcasted_iota(jnp.int32, (QT, QT), 1)
    e = jnp.exp(jnp.where((c < r) | (ri >= ci), s, -1e30))
    lrow = jnp.sum(e, axis=-1, keepdims=True)
    pv = jnp.dot(e, vb, preferred_element_type=jnp.float32)

    @pl.when(c == 0)
    def _():
        l_scr[...] = lrow
        o_ref[0] = pv

    @pl.when(c != 0)
    def _():
        l_scr[...] = l_scr[...] + lrow
        o_ref[0] = o_ref[0] + pv

    @pl.when(c == r)
    def _():
        o_ref[0] = o_ref[0] * (1.0 / l_scr[...])


def _k2(rmap, cmap, q3, k3, v3):
    grid_spec = pltpu.PrefetchScalarGridSpec(
        num_scalar_prefetch=2,
        grid=(NH, NV),
        in_specs=[
            pl.BlockSpec((1, QT, HD), lambda h, i, rm, cm: (h, rm[i], 0)),
            pl.BlockSpec((1, SEQ, HD), lambda h, i, rm, cm: (h, 0, 0)),
            pl.BlockSpec((1, SEQ, HD), lambda h, i, rm, cm: (h, 0, 0)),
        ],
        out_specs=pl.BlockSpec((1, QT, HD), lambda h, i, rm, cm: (h, rm[i], 0)),
        scratch_shapes=[pltpu.VMEM((QT, 1), jnp.float32)],
    )
    return pl.pallas_call(
        _attn_body,
        grid_spec=grid_spec,
        out_shape=jax.ShapeDtypeStruct((NH, SEQ, HD), jnp.float32),
    )(rmap, cmap, q3, k3, v3)


def _k3_body(ao_ref, hs_ref, pw_ref, g_ref, b_ref, rw_ref,
             ha_ref, flat_ref, eidx_ref, wgt_ref):
    ao = jnp.concatenate([ao_ref[h] for h in range(NH)], axis=1)
    proj = lax.dot_general(ao, pw_ref[...], (((1,), (1,)), ((), ())),
                           preferred_element_type=jnp.float32)
    ha = hs_ref[...] + proj
    ha_ref[...] = ha
    mu = jnp.mean(ha, axis=-1, keepdims=True)
    var = jnp.mean((ha - mu) ** 2, axis=-1, keepdims=True)
    ln = (ha - mu) * lax.rsqrt(var + 1e-5) * g_ref[...] + b_ref[...]
    flat_ref[...] = ln
    logits = jnp.dot(ln, rw_ref[...], preferred_element_type=jnp.float32)
    mx = jnp.max(logits, axis=-1, keepdims=True)
    ex = jnp.exp(logits - mx)
    probs = ex / jnp.sum(ex, axis=-1, keepdims=True)
    col = lax.broadcasted_iota(jnp.int32, (ST, E), 1)
    m1 = jnp.max(probs, axis=-1, keepdims=True)
    e1 = jnp.min(jnp.where(probs == m1, col, E), axis=-1, keepdims=True)
    masked = jnp.where(col == e1, -1.0, probs)
    m2 = jnp.max(masked, axis=-1, keepdims=True)
    e2 = jnp.min(jnp.where(masked == m2, col, E), axis=-1, keepdims=True)
    s = m1 + m2
    zi = jnp.zeros((ST, E - 2), jnp.int32)
    zf = jnp.zeros((ST, E - 2), jnp.float32)
    eidx_ref[...] = jnp.concatenate([e1, e2, zi], axis=1)
    wgt_ref[...] = jnp.concatenate([m1 / s, m2 / s, zf], axis=1)


def _k3(ao, hs2d, pw, ln2w, ln2b, rw):
    return pl.pallas_call(
        _k3_body,
        grid=(SEQ // ST,),
        in_specs=[
            pl.BlockSpec((NH, ST, HD), lambda i: (0, i, 0)),
            pl.BlockSpec((ST, HID), lambda i: (i, 0)),
            pl.BlockSpec((HID, HID), lambda i: (0, 0)),
            pl.BlockSpec((1, HID), lambda i: (0, 0)),
            pl.BlockSpec((1, HID), lambda i: (0, 0)),
            pl.BlockSpec((HID, E), lambda i: (0, 0)),
        ],
        out_specs=[
            pl.BlockSpec((ST, HID), lambda i: (i, 0)),
            pl.BlockSpec((ST, HID), lambda i: (i, 0)),
            pl.BlockSpec((ST, E), lambda i: (i, 0)),
            pl.BlockSpec((ST, E), lambda i: (i, 0)),
        ],
        out_shape=[
            jax.ShapeDtypeStruct((SEQ, HID), jnp.float32),
            jax.ShapeDtypeStruct((SEQ, HID), jnp.float32),
            jax.ShapeDtypeStruct((SEQ, E), jnp.int32),
            jax.ShapeDtypeStruct((SEQ, E), jnp.float32),
        ],
    )(ao, hs2d, pw, ln2w, ln2b, rw)


BR = 256


def _k3r_body(eidx_ref, pos0_ref, pos1_ref, te_ref):
    x = eidx_ref[...]
    laneE = lax.broadcasted_iota(jnp.int32, (SEQ, E), 1)
    m0 = (x[:, 0:1] == laneE).astype(jnp.float32)
    m1 = (x[:, 1:2] == laneE).astype(jnp.float32)
    m = jnp.concatenate([m0, m1], axis=1)

    ri = lax.broadcasted_iota(jnp.int32, (BR, BR), 0)
    ci = lax.broadcasted_iota(jnp.int32, (BR, BR), 1)
    tri = (ci < ri).astype(jnp.float32)

    carry = jnp.zeros((1, 2 * E), jnp.float32)
    blocks = []
    for b in range(SEQ // BR):
        mb = m[b * BR:(b + 1) * BR]
        blocks.append(jnp.dot(tri, mb, preferred_element_type=jnp.float32)
                      + carry)
        carry = carry + jnp.sum(mb, axis=0, keepdims=True)
    s = jnp.concatenate(blocks, axis=0)

    ct0 = carry[:, :E]
    ctot = carry[:, :E] + carry[:, E:]
    psi = ((ctot.astype(jnp.int32) + (ET - 1)) // ET) * ET
    ps = psi.astype(jnp.float32)
    i8 = lax.broadcasted_iota(jnp.int32, (E, E), 0)
    j8 = lax.broadcasted_iota(jnp.int32, (E, E), 1)
    tri8 = (i8 < j8).astype(jnp.float32)
    off = jnp.dot(ps, tri8, preferred_element_type=jnp.float32)

    pos0 = jnp.sum(m0 * (off + s[:, :E]), axis=1, keepdims=True)
    pos1 = jnp.sum(m1 * (off + ct0 + s[:, E:]), axis=1, keepdims=True)
    pos0_ref[...] = pos0.astype(jnp.int32)
    pos1_ref[...] = pos1.astype(jnp.int32)

    tv = (lax.broadcasted_iota(jnp.int32, (NTE, E), 0) * ET).astype(jnp.float32)
    cnt = jnp.sum((off <= tv).astype(jnp.int32), axis=1, keepdims=True) - 1
    nvalid = jnp.sum(psi, axis=1, keepdims=True) // ET
    idxn = lax.broadcasted_iota(jnp.int32, (NTE, 1), 0)
    te_ref[...] = jnp.where(idxn == NT, nvalid, jnp.clip(cnt, 0, E - 1))


def _k3r(eidx):
    return pl.pallas_call(
        _k3r_body,
        in_specs=[pl.BlockSpec((SEQ, E), lambda: (0, 0))],
        out_specs=[
            pl.BlockSpec((SEQ, 1), lambda: (0, 0)),
            pl.BlockSpec((SEQ, 1), lambda: (0, 0)),
            pl.BlockSpec((NTE, 1), lambda: (0, 0)),
        ],
        out_shape=[
            jax.ShapeDtypeStruct((SEQ, 1), jnp.int32),
            jax.ShapeDtypeStruct((SEQ, 1), jnp.int32),
            jax.ShapeDtypeStruct((NTE, 1), jnp.int32),
        ],
    )(eidx)


def _sca_body(pos0_hbm, pos1_hbm, flat_hbm, xp_hbm,
              posb0, posb1, rowbuf, sem):
    wid = lax.axis_index("s") * NC + lax.axis_index("c")
    tok_base = wid * TPW
    pltpu.sync_copy(pos0_hbm.at[pl.ds(tok_base, TPW)], posb0)
    pltpu.sync_copy(pos1_hbm.at[pl.ds(tok_base, TPW)], posb1)
    for g in range(TPW // L):
        pltpu.sync_copy(flat_hbm.at[pl.ds(tok_base + g * L, L)], rowbuf)
        p0 = posb0[pl.ds(g * L, L)]
        pltpu.async_copy(rowbuf, xp_hbm.at[p0], sem).wait()
        p1 = posb1[pl.ds(g * L, L)]
        pltpu.async_copy(rowbuf, xp_hbm.at[p1], sem).wait()


def _sc_scatter(pos0, pos1, flat):
    mesh = plsc.VectorSubcoreMesh(core_axis_name="c", subcore_axis_name="s",
                                  num_cores=NC, num_subcores=NS)
    f = pl.kernel(
        _sca_body,
        out_type=jax.ShapeDtypeStruct((P, HID), jnp.float32),
        mesh=mesh,
        scratch_types=[
            pltpu.VMEM((TPW,), jnp.int32),
            pltpu.VMEM((TPW,), jnp.int32),
            pltpu.VMEM((L, HID), jnp.float32),
            pltpu.SemaphoreType.DMA,
        ],
    )
    return f(pos0, pos1, flat)


def _k4_body(te_ref, x_ref, w1_ref, w2_ref, y_ref, w1b, w2b):
    i = pl.program_id(0)

    @pl.when(i < te_ref[NT])
    def _():
        prev = te_ref[jnp.maximum(i - 1, 0)]

        @pl.when((i == 0) | (te_ref[i] != prev))
        def _():
            w1b[...] = w1_ref[0].astype(jnp.bfloat16)
            w2b[...] = w2_ref[0].astype(jnp.bfloat16)

        xb = x_ref[...].astype(jnp.bfloat16)
        h = jnp.dot(xb, w1b[...], preferred_element_type=jnp.float32)
        h = jax.nn.gelu(h)
        y_ref[...] = jnp.dot(h.astype(jnp.bfloat16), w2b[...],
                             preferred_element_type=jnp.float32)


def _k4(te, xp, w1, w2):
    grid_spec = pltpu.PrefetchScalarGridSpec(
        num_scalar_prefetch=1,
        grid=(NT,),
        in_specs=[
            pl.BlockSpec((ET, HID), lambda i, te: (i, 0)),
            pl.BlockSpec((1, HID, DFF), lambda i, te: (te[i], 0, 0)),
            pl.BlockSpec((1, DFF, HID), lambda i, te: (te[i], 0, 0)),
        ],
        out_specs=pl.BlockSpec((ET, HID), lambda i, te: (i, 0)),
        scratch_shapes=[
            pltpu.VMEM((HID, DFF), jnp.bfloat16),
            pltpu.VMEM((DFF, HID), jnp.bfloat16),
        ],
    )
    return pl.pallas_call(
        _k4_body,
        grid_spec=grid_spec,
        out_shape=jax.ShapeDtypeStruct((P, HID), jnp.float32),
    )(te, xp, w1, w2)


def _scb_body(yp_hbm, pos0_hbm, pos1_hbm, y0_hbm, y1_hbm,
              idxv, rows, sem):
    wid = lax.axis_index("s") * NC + lax.axis_index("c")
    base = wid * TPW
    pltpu.sync_copy(pos0_hbm.at[pl.ds(base, TPW)], idxv)
    pltpu.async_copy(yp_hbm.at[idxv], rows, sem).wait()
    pltpu.sync_copy(rows, y0_hbm.at[pl.ds(base, TPW)])
    pltpu.sync_copy(pos1_hbm.at[pl.ds(base, TPW)], idxv)
    pltpu.async_copy(yp_hbm.at[idxv], rows, sem).wait()
    pltpu.sync_copy(rows, y1_hbm.at[pl.ds(base, TPW)])


def _sc_gather(yp, pos0, pos1):
    mesh = plsc.VectorSubcoreMesh(core_axis_name="c", subcore_axis_name="s", num_cores=NC, num_subcores=NS)
    f = pl.kernel(
        _scb_body,
        out_type=[
            jax.ShapeDtypeStruct((SEQ, HID), jnp.float32),
            jax.ShapeDtypeStruct((SEQ, HID), jnp.float32),
        ],
        mesh=mesh,
        scratch_types=[
            pltpu.VMEM((TPW,), jnp.int32),
            pltpu.VMEM((TPW, HID), jnp.float32),
            pltpu.SemaphoreType.DMA,
        ],
    )
    return f(yp, pos0, pos1)


def _k5_body(ha_ref, y0_ref, y1_ref, w0_ref, w1_ref, o_ref):
    o_ref[...] = (ha_ref[...] + w0_ref[...] * y0_ref[...]
                  + w1_ref[...] * y1_ref[...])


def _k5(ha, y0, y1, w0, w1):
    return pl.pallas_call(
        _k5_body,
        grid=(SEQ // ST,),
        in_specs=[
            pl.BlockSpec((ST, HID), lambda i: (i, 0)),
            pl.BlockSpec((ST, HID), lambda i: (i, 0)),
            pl.BlockSpec((ST, HID), lambda i: (i, 0)),
            pl.BlockSpec((ST, 1), lambda i: (i, 0)),
            pl.BlockSpec((ST, 1), lambda i: (i, 0)),
        ],
        out_specs=pl.BlockSpec((ST, HID), lambda i: (i, 0)),
        out_shape=jax.ShapeDtypeStruct((SEQ, HID), jnp.float32),
    )(ha, y0, y1, w0, w1)


def kernel(hidden_states, ln1_weight, ln1_bias, ln2_weight, ln2_bias,
           qkv_weight, proj_weight, router_weight, moe_w1, moe_w2):
    hs2d = hidden_states.reshape(SEQ, HID)
    w3 = qkv_weight.reshape(NH, 3, HD, HID)
    wq = w3[:, 0].reshape(NH * HD, HID).T
    wk = w3[:, 1].reshape(NH * HD, HID).T
    wv = w3[:, 2].reshape(NH * HD, HID).T
    q3, k3, v3 = _k1(hs2d, ln1_weight.reshape(1, HID),
                     ln1_bias.reshape(1, HID), wq, wk, wv)
    rmap = jnp.asarray([r for r in range(NQ) for c in range(r + 1)],
                       dtype=jnp.int32)
    cmap = jnp.asarray([c for r in range(NQ) for c in range(r + 1)],
                       dtype=jnp.int32)
    ao = _k2(rmap, cmap, q3, k3, v3)
    ha, flat, eidx, wgt = _k3(ao, hs2d, proj_weight,
                              ln2_weight.reshape(1, HID),
                              ln2_bias.reshape(1, HID), router_weight)
    pos0_2d, pos1_2d, te_2d = _k3r(eidx)
    pos0 = pos0_2d.reshape(SEQ)
    pos1 = pos1_2d.reshape(SEQ)
    te = te_2d.reshape(NTE)
    xp = _sc_scatter(pos0, pos1, flat)
    yp = _k4(te, xp, moe_w1, moe_w2)
    y0, y1 = _sc_gather(yp, pos0, pos1)
    out = _k5(ha, y0, y1, wgt[:, 0:1], wgt[:, 1:2])
    return out.reshape(SEQ, 1, HID)


# speedup vs baseline: 1.4727x; 1.4727x over previous
import functools

import jax
import jax.numpy as jnp
from jax import lax
from jax.experimental import pallas as pl
from jax.experimental.pallas import tpu as pltpu
from jax.experimental.pallas import tpu_sc as plsc

SEQ, HID = 2048, 1024
NH, HD = 16, 64
E, TOPK, DFF = 8, 2, 2048
QKV_OUT = (NH + 2 * NH) * HD

ST = 256
QT = 256
ET = 128
NPAIR = SEQ * TOPK
P = 5120
NT = P // ET
NTE = 48

NC, NS, L = 2, 16, 16
NW = NC * NS
TPW = SEQ // NW


def _k1_body(x_ref, g_ref, b_ref, wq_ref, wk_ref, wv_ref,
             q_ref, k_ref, v_ref):
    x = x_ref[...]
    mu = jnp.mean(x, axis=-1, keepdims=True)
    var = jnp.mean((x - mu) ** 2, axis=-1, keepdims=True)
    ln = (x - mu) * lax.rsqrt(var + 1e-5) * g_ref[...] + b_ref[...]
    qa = jnp.dot(ln, wq_ref[...], preferred_element_type=jnp.float32)
    ka = jnp.dot(ln, wk_ref[...], preferred_element_type=jnp.float32)
    va = jnp.dot(ln, wv_ref[...], preferred_element_type=jnp.float32)
    for h in range(NH):
        q_ref[h] = qa[:, HD * h:HD * (h + 1)]
        k_ref[h] = ka[:, HD * h:HD * (h + 1)]
        v_ref[h] = va[:, HD * h:HD * (h + 1)]


def _k1(hs2d, ln1w, ln1b, wq, wk, wv):
    hspec = pl.BlockSpec((NH, ST, HD), lambda i: (0, i, 0))
    return pl.pallas_call(
        _k1_body,
        grid=(SEQ // ST,),
        in_specs=[
            pl.BlockSpec((ST, HID), lambda i: (i, 0)),
            pl.BlockSpec((1, HID), lambda i: (0, 0)),
            pl.BlockSpec((1, HID), lambda i: (0, 0)),
            pl.BlockSpec((HID, HID), lambda i: (0, 0)),
            pl.BlockSpec((HID, HID), lambda i: (0, 0)),
            pl.BlockSpec((HID, HID), lambda i: (0, 0)),
        ],
        out_specs=[hspec, hspec, hspec],
        out_shape=[jax.ShapeDtypeStruct((NH, SEQ, HD), jnp.float32)] * 3,
    )(hs2d, ln1w, ln1b, wq, wk, wv)


def _attn_body(q_ref, k_ref, v_ref, o_ref):
    r = pl.program_id(1)
    scale = 1.0 / (HD ** 0.5)
    q = q_ref[0] * scale
    s = lax.dot_general(q, k_ref[0], (((1,), (1,)), ((), ())),
                        preferred_element_type=jnp.float32)
    ri = lax.broadcasted_iota(jnp.int32, (QT, SEQ), 0) + r * QT
    ci = lax.broadcasted_iota(jnp.int32, (QT, SEQ), 1)
    p = jnp.exp(jnp.where(ri >= ci, s, -1e30))
    l = jnp.sum(p, axis=-1, keepdims=True)
    o_ref[0] = jnp.dot(p, v_ref[0],
                       preferred_element_type=jnp.float32) * (1.0 / l)


def _k2(q3, k3, v3):
    return pl.pallas_call(
        _attn_body,
        grid=(NH, SEQ // QT),
        in_specs=[
            pl.BlockSpec((1, QT, HD), lambda h, r: (h, r, 0)),
            pl.BlockSpec((1, SEQ, HD), lambda h, r: (h, 0, 0)),
            pl.BlockSpec((1, SEQ, HD), lambda h, r: (h, 0, 0)),
        ],
        out_specs=pl.BlockSpec((1, QT, HD), lambda h, r: (h, r, 0)),
        out_shape=jax.ShapeDtypeStruct((NH, SEQ, HD), jnp.float32),
    )(q3, k3, v3)


def _k3_body(ao_ref, hs_ref, pw_ref, g_ref, b_ref, rw_ref,
             ha_ref, flat_ref, eidx_ref, wgt_ref):
    ao = jnp.concatenate([ao_ref[h] for h in range(NH)], axis=1)
    proj = lax.dot_general(ao, pw_ref[...], (((1,), (1,)), ((), ())),
                           preferred_element_type=jnp.float32)
    ha = hs_ref[...] + proj
    ha_ref[...] = ha
    mu = jnp.mean(ha, axis=-1, keepdims=True)
    var = jnp.mean((ha - mu) ** 2, axis=-1, keepdims=True)
    ln = (ha - mu) * lax.rsqrt(var + 1e-5) * g_ref[...] + b_ref[...]
    flat_ref[...] = ln
    logits = jnp.dot(ln, rw_ref[...], preferred_element_type=jnp.float32)
    mx = jnp.max(logits, axis=-1, keepdims=True)
    ex = jnp.exp(logits - mx)
    probs = ex / jnp.sum(ex, axis=-1, keepdims=True)
    col = lax.broadcasted_iota(jnp.int32, (ST, E), 1)
    m1 = jnp.max(probs, axis=-1, keepdims=True)
    e1 = jnp.min(jnp.where(probs == m1, col, E), axis=-1, keepdims=True)
    masked = jnp.where(col == e1, -1.0, probs)
    m2 = jnp.max(masked, axis=-1, keepdims=True)
    e2 = jnp.min(jnp.where(masked == m2, col, E), axis=-1, keepdims=True)
    s = m1 + m2
    zi = jnp.zeros((ST, E - 2), jnp.int32)
    zf = jnp.zeros((ST, E - 2), jnp.float32)
    eidx_ref[...] = jnp.concatenate([e1, e2, zi], axis=1)
    wgt_ref[...] = jnp.concatenate([m1 / s, m2 / s, zf], axis=1)


def _k3(ao, hs2d, pw, ln2w, ln2b, rw):
    return pl.pallas_call(
        _k3_body,
        grid=(SEQ // ST,),
        in_specs=[
            pl.BlockSpec((NH, ST, HD), lambda i: (0, i, 0)),
            pl.BlockSpec((ST, HID), lambda i: (i, 0)),
            pl.BlockSpec((HID, HID), lambda i: (0, 0)),
            pl.BlockSpec((1, HID), lambda i: (0, 0)),
            pl.BlockSpec((1, HID), lambda i: (0, 0)),
            pl.BlockSpec((HID, E), lambda i: (0, 0)),
        ],
        out_specs=[
            pl.BlockSpec((ST, HID), lambda i: (i, 0)),
            pl.BlockSpec((ST, HID), lambda i: (i, 0)),
            pl.BlockSpec((ST, E), lambda i: (i, 0)),
            pl.BlockSpec((ST, E), lambda i: (i, 0)),
        ],
        out_shape=[
            jax.ShapeDtypeStruct((SEQ, HID), jnp.float32),
            jax.ShapeDtypeStruct((SEQ, HID), jnp.float32),
            jax.ShapeDtypeStruct((SEQ, E), jnp.int32),
            jax.ShapeDtypeStruct((SEQ, E), jnp.float32),
        ],
    )(ao, hs2d, pw, ln2w, ln2b, rw)


BR = 256


def _k3r_body(eidx_ref, pos0_ref, pos1_ref, te_ref):
    x = eidx_ref[...]
    laneE = lax.broadcasted_iota(jnp.int32, (SEQ, E), 1)
    m0 = (x[:, 0:1] == laneE).astype(jnp.float32)
    m1 = (x[:, 1:2] == laneE).astype(jnp.float32)
    m = jnp.concatenate([m0, m1], axis=1)

    ri = lax.broadcasted_iota(jnp.int32, (BR, BR), 0)
    ci = lax.broadcasted_iota(jnp.int32, (BR, BR), 1)
    tri = (ci < ri).astype(jnp.float32)

    carry = jnp.zeros((1, 2 * E), jnp.float32)
    blocks = []
    for b in range(SEQ // BR):
        mb = m[b * BR:(b + 1) * BR]
        blocks.append(jnp.dot(tri, mb, preferred_element_type=jnp.float32)
                      + carry)
        carry = carry + jnp.sum(mb, axis=0, keepdims=True)
    s = jnp.concatenate(blocks, axis=0)

    ct0 = carry[:, :E]
    ctot = carry[:, :E] + carry[:, E:]
    psi = ((ctot.astype(jnp.int32) + (ET - 1)) // ET) * ET
    ps = psi.astype(jnp.float32)
    i8 = lax.broadcasted_iota(jnp.int32, (E, E), 0)
    j8 = lax.broadcasted_iota(jnp.int32, (E, E), 1)
    tri8 = (i8 < j8).astype(jnp.float32)
    off = jnp.dot(ps, tri8, preferred_element_type=jnp.float32)

    pos0 = jnp.sum(m0 * (off + s[:, :E]), axis=1, keepdims=True)
    pos1 = jnp.sum(m1 * (off + ct0 + s[:, E:]), axis=1, keepdims=True)
    pos0_ref[...] = pos0.astype(jnp.int32)
    pos1_ref[...] = pos1.astype(jnp.int32)

    tv = (lax.broadcasted_iota(jnp.int32, (NTE, E), 0) * ET).astype(jnp.float32)
    cnt = jnp.sum((off <= tv).astype(jnp.int32), axis=1, keepdims=True) - 1
    nvalid = jnp.sum(psi, axis=1, keepdims=True) // ET
    idxn = lax.broadcasted_iota(jnp.int32, (NTE, 1), 0)
    te_ref[...] = jnp.where(idxn == NT, nvalid, jnp.clip(cnt, 0, E - 1))


def _k3r(eidx):
    return pl.pallas_call(
        _k3r_body,
        in_specs=[pl.BlockSpec((SEQ, E), lambda: (0, 0))],
        out_specs=[
            pl.BlockSpec((SEQ, 1), lambda: (0, 0)),
            pl.BlockSpec((SEQ, 1), lambda: (0, 0)),
            pl.BlockSpec((NTE, 1), lambda: (0, 0)),
        ],
        out_shape=[
            jax.ShapeDtypeStruct((SEQ, 1), jnp.int32),
            jax.ShapeDtypeStruct((SEQ, 1), jnp.int32),
            jax.ShapeDtypeStruct((NTE, 1), jnp.int32),
        ],
    )(eidx)


def _sca_body(pos0_hbm, pos1_hbm, flat_hbm, xp_hbm,
              posb0, posb1, rowbuf, sem):
    wid = lax.axis_index("s") * NC + lax.axis_index("c")
    tok_base = wid * TPW
    pltpu.sync_copy(pos0_hbm.at[pl.ds(tok_base, TPW)], posb0)
    pltpu.sync_copy(pos1_hbm.at[pl.ds(tok_base, TPW)], posb1)
    for g in range(TPW // L):
        pltpu.sync_copy(flat_hbm.at[pl.ds(tok_base + g * L, L)], rowbuf)
        p0 = posb0[pl.ds(g * L, L)]
        pltpu.async_copy(rowbuf, xp_hbm.at[p0], sem).wait()
        p1 = posb1[pl.ds(g * L, L)]
        pltpu.async_copy(rowbuf, xp_hbm.at[p1], sem).wait()


def _sc_scatter(pos0, pos1, flat):
    mesh = plsc.VectorSubcoreMesh(core_axis_name="c", subcore_axis_name="s",
                                  num_cores=NC, num_subcores=NS)
    f = pl.kernel(
        _sca_body,
        out_type=jax.ShapeDtypeStruct((P, HID), jnp.float32),
        mesh=mesh,
        scratch_types=[
            pltpu.VMEM((TPW,), jnp.int32),
            pltpu.VMEM((TPW,), jnp.int32),
            pltpu.VMEM((L, HID), jnp.float32),
            pltpu.SemaphoreType.DMA,
        ],
    )
    return f(pos0, pos1, flat)


def _k4_body(te_ref, x_ref, w1_ref, w2_ref, y_ref, w1b, w2b):
    i = pl.program_id(0)

    @pl.when(i < te_ref[NT])
    def _():
        prev = te_ref[jnp.maximum(i - 1, 0)]

        @pl.when((i == 0) | (te_ref[i] != prev))
        def _():
            w1b[...] = w1_ref[0].astype(jnp.bfloat16)
            w2b[...] = w2_ref[0].astype(jnp.bfloat16)

        xb = x_ref[...].astype(jnp.bfloat16)
        h = jnp.dot(xb, w1b[...], preferred_element_type=jnp.float32)
        h = jax.nn.gelu(h)
        y_ref[...] = jnp.dot(h.astype(jnp.bfloat16), w2b[...],
                             preferred_element_type=jnp.float32)


def _k4(te, xp, w1, w2):
    grid_spec = pltpu.PrefetchScalarGridSpec(
        num_scalar_prefetch=1,
        grid=(NT,),
        in_specs=[
            pl.BlockSpec((ET, HID), lambda i, te: (i, 0)),
            pl.BlockSpec((1, HID, DFF), lambda i, te: (te[i], 0, 0)),
            pl.BlockSpec((1, DFF, HID), lambda i, te: (te[i], 0, 0)),
        ],
        out_specs=pl.BlockSpec((ET, HID), lambda i, te: (i, 0)),
        scratch_shapes=[
            pltpu.VMEM((HID, DFF), jnp.bfloat16),
            pltpu.VMEM((DFF, HID), jnp.bfloat16),
        ],
    )
    return pl.pallas_call(
        _k4_body,
        grid_spec=grid_spec,
        out_shape=jax.ShapeDtypeStruct((P, HID), jnp.float32),
    )(te, xp, w1, w2)


def _scb_body(yp_hbm, pos0_hbm, pos1_hbm, y0_hbm, y1_hbm,
              idxv, rows, sem):
    wid = lax.axis_index("s") * NC + lax.axis_index("c")
    base = wid * TPW
    pltpu.sync_copy(pos0_hbm.at[pl.ds(base, TPW)], idxv)
    pltpu.async_copy(yp_hbm.at[idxv], rows, sem).wait()
    pltpu.sync_copy(rows, y0_hbm.at[pl.ds(base, TPW)])
    pltpu.sync_copy(pos1_hbm.at[pl.ds(base, TPW)], idxv)
    pltpu.async_copy(yp_hbm.at[idxv], rows, sem).wait()
    pltpu.sync_copy(rows, y1_hbm.at[pl.ds(base, TPW)])


def _sc_gather(yp, pos0, pos1):
    mesh = plsc.VectorSubcoreMesh(core_axis_name="c", subcore_axis_name="s", num_cores=NC, num_subcores=NS)
    f = pl.kernel(
        _scb_body,
        out_type=[
            jax.ShapeDtypeStruct((SEQ, HID), jnp.float32),
            jax.ShapeDtypeStruct((SEQ, HID), jnp.float32),
        ],
        mesh=mesh,
        scratch_types=[
            pltpu.VMEM((TPW,), jnp.int32),
            pltpu.VMEM((TPW, HID), jnp.float32),
            pltpu.SemaphoreType.DMA,
        ],
    )
    return f(yp, pos0, pos1)


def _k5_body(ha_ref, y0_ref, y1_ref, w0_ref, w1_ref, o_ref):
    o_ref[...] = (ha_ref[...] + w0_ref[...] * y0_ref[...]
                  + w1_ref[...] * y1_ref[...])


def _k5(ha, y0, y1, w0, w1):
    return pl.pallas_call(
        _k5_body,
        grid=(SEQ // ST,),
        in_specs=[
            pl.BlockSpec((ST, HID), lambda i: (i, 0)),
            pl.BlockSpec((ST, HID), lambda i: (i, 0)),
            pl.BlockSpec((ST, HID), lambda i: (i, 0)),
            pl.BlockSpec((ST, 1), lambda i: (i, 0)),
            pl.BlockSpec((ST, 1), lambda i: (i, 0)),
        ],
        out_specs=pl.BlockSpec((ST, HID), lambda i: (i, 0)),
        out_shape=jax.ShapeDtypeStruct((SEQ, HID), jnp.float32),
    )(ha, y0, y1, w0, w1)


def kernel(hidden_states, ln1_weight, ln1_bias, ln2_weight, ln2_bias,
           qkv_weight, proj_weight, router_weight, moe_w1, moe_w2):
    hs2d = hidden_states.reshape(SEQ, HID)
    w3 = qkv_weight.reshape(NH, 3, HD, HID)
    wq = w3[:, 0].reshape(NH * HD, HID).T
    wk = w3[:, 1].reshape(NH * HD, HID).T
    wv = w3[:, 2].reshape(NH * HD, HID).T
    q3, k3, v3 = _k1(hs2d, ln1_weight.reshape(1, HID),
                     ln1_bias.reshape(1, HID), wq, wk, wv)
    ao = _k2(q3, k3, v3)
    ha, flat, eidx, wgt = _k3(ao, hs2d, proj_weight,
                              ln2_weight.reshape(1, HID),
                              ln2_bias.reshape(1, HID), router_weight)
    pos0_2d, pos1_2d, te_2d = _k3r(eidx)
    pos0 = pos0_2d.reshape(SEQ)
    pos1 = pos1_2d.reshape(SEQ)
    te = te_2d.reshape(NTE)
    xp = _sc_scatter(pos0, pos1, flat)
    yp = _k4(te, xp, moe_w1, moe_w2)
    y0, y1 = _sc_gather(yp, pos0, pos1)
    out = _k5(ha, y0, y1, wgt[:, 0:1], wgt[:, 1:2])
    return out.reshape(SEQ, 1, HID)


# speedup vs baseline: 1.4737x; 1.0007x over previous
import functools

import jax
import jax.numpy as jnp
from jax import lax
from jax.experimental import pallas as pl
from jax.experimental.pallas import tpu as pltpu
from jax.experimental.pallas import tpu_sc as plsc

SEQ, HID = 2048, 1024
NH, HD = 16, 64
E, TOPK, DFF = 8, 2, 2048
QKV_OUT = (NH + 2 * NH) * HD

ST = 256
QT = 256
ET = 128
NPAIR = SEQ * TOPK
P = 5120
NT = P // ET
NTE = 48

NC, NS, L = 2, 16, 16
NW = NC * NS
TPW = SEQ // NW


def _k1_body(x_ref, g_ref, b_ref, wq_ref, wk_ref, wv_ref,
             q_ref, k_ref, v_ref):
    x = x_ref[...]
    mu = jnp.mean(x, axis=-1, keepdims=True)
    var = jnp.mean((x - mu) ** 2, axis=-1, keepdims=True)
    ln = (x - mu) / jnp.sqrt(var + 1e-5) * g_ref[...] + b_ref[...]
    qa = jnp.dot(ln, wq_ref[...], preferred_element_type=jnp.float32)
    ka = jnp.dot(ln, wk_ref[...], preferred_element_type=jnp.float32)
    va = jnp.dot(ln, wv_ref[...], preferred_element_type=jnp.float32)
    for h in range(NH):
        q_ref[h] = qa[:, HD * h:HD * (h + 1)]
        k_ref[h] = ka[:, HD * h:HD * (h + 1)]
        v_ref[h] = va[:, HD * h:HD * (h + 1)]


def _k1(hs2d, ln1w, ln1b, wq, wk, wv):
    hspec = pl.BlockSpec((NH, ST, HD), lambda i: (0, i, 0))
    return pl.pallas_call(
        _k1_body,
        grid=(SEQ // ST,),
        in_specs=[
            pl.BlockSpec((ST, HID), lambda i: (i, 0)),
            pl.BlockSpec((1, HID), lambda i: (0, 0)),
            pl.BlockSpec((1, HID), lambda i: (0, 0)),
            pl.BlockSpec((HID, HID), lambda i: (0, 0)),
            pl.BlockSpec((HID, HID), lambda i: (0, 0)),
            pl.BlockSpec((HID, HID), lambda i: (0, 0)),
        ],
        out_specs=[hspec, hspec, hspec],
        out_shape=[jax.ShapeDtypeStruct((NH, SEQ, HD), jnp.float32)] * 3,
    )(hs2d, ln1w, ln1b, wq, wk, wv)


def _attn_body(q_ref, k_ref, v_ref, o_ref):
    r = pl.program_id(1)
    scale = 1.0 / (HD ** 0.5)
    s = lax.dot_general(q_ref[0], k_ref[0], (((1,), (1,)), ((), ())),
                        preferred_element_type=jnp.float32) * scale
    ri = lax.broadcasted_iota(jnp.int32, (QT, SEQ), 0) + r * QT
    ci = lax.broadcasted_iota(jnp.int32, (QT, SEQ), 1)
    p = jnp.exp(jnp.where(ri >= ci, s, -1e30))
    l = jnp.sum(p, axis=-1, keepdims=True)
    o_ref[0] = jnp.dot(p, v_ref[0],
                       preferred_element_type=jnp.float32) * (1.0 / l)


def _k2(q3, k3, v3):
    return pl.pallas_call(
        _attn_body,
        grid=(NH, SEQ // QT),
        in_specs=[
            pl.BlockSpec((1, QT, HD), lambda h, r: (h, r, 0)),
            pl.BlockSpec((1, SEQ, HD), lambda h, r: (h, 0, 0)),
            pl.BlockSpec((1, SEQ, HD), lambda h, r: (h, 0, 0)),
        ],
        out_specs=pl.BlockSpec((1, QT, HD), lambda h, r: (h, r, 0)),
        out_shape=jax.ShapeDtypeStruct((NH, SEQ, HD), jnp.float32),
    )(q3, k3, v3)


def _k3_body(ao_ref, hs_ref, pw_ref, g_ref, b_ref, rw_ref,
             ha_ref, flat_ref, eidx_ref, wgt_ref):
    ao = jnp.concatenate([ao_ref[h] for h in range(NH)], axis=1)
    proj = lax.dot_general(ao, pw_ref[...], (((1,), (1,)), ((), ())),
                           preferred_element_type=jnp.float32)
    ha = hs_ref[...] + proj
    ha_ref[...] = ha
    mu = jnp.mean(ha, axis=-1, keepdims=True)
    var = jnp.mean((ha - mu) ** 2, axis=-1, keepdims=True)
    ln = (ha - mu) / jnp.sqrt(var + 1e-5) * g_ref[...] + b_ref[...]
    flat_ref[...] = ln
    logits = jnp.dot(ln, rw_ref[...], preferred_element_type=jnp.float32)
    mx = jnp.max(logits, axis=-1, keepdims=True)
    ex = jnp.exp(logits - mx)
    probs = ex / jnp.sum(ex, axis=-1, keepdims=True)
    col = lax.broadcasted_iota(jnp.int32, (ST, E), 1)
    m1 = jnp.max(probs, axis=-1, keepdims=True)
    e1 = jnp.min(jnp.where(probs == m1, col, E), axis=-1, keepdims=True)
    masked = jnp.where(col == e1, -1.0, probs)
    m2 = jnp.max(masked, axis=-1, keepdims=True)
    e2 = jnp.min(jnp.where(masked == m2, col, E), axis=-1, keepdims=True)
    s = m1 + m2
    zi = jnp.zeros((ST, E - 2), jnp.int32)
    zf = jnp.zeros((ST, E - 2), jnp.float32)
    eidx_ref[...] = jnp.concatenate([e1, e2, zi], axis=1)
    wgt_ref[...] = jnp.concatenate([m1 / s, m2 / s, zf], axis=1)


def _k3(ao, hs2d, pw, ln2w, ln2b, rw):
    return pl.pallas_call(
        _k3_body,
        grid=(SEQ // ST,),
        in_specs=[
            pl.BlockSpec((NH, ST, HD), lambda i: (0, i, 0)),
            pl.BlockSpec((ST, HID), lambda i: (i, 0)),
            pl.BlockSpec((HID, HID), lambda i: (0, 0)),
            pl.BlockSpec((1, HID), lambda i: (0, 0)),
            pl.BlockSpec((1, HID), lambda i: (0, 0)),
            pl.BlockSpec((HID, E), lambda i: (0, 0)),
        ],
        out_specs=[
            pl.BlockSpec((ST, HID), lambda i: (i, 0)),
            pl.BlockSpec((ST, HID), lambda i: (i, 0)),
            pl.BlockSpec((ST, E), lambda i: (i, 0)),
            pl.BlockSpec((ST, E), lambda i: (i, 0)),
        ],
        out_shape=[
            jax.ShapeDtypeStruct((SEQ, HID), jnp.float32),
            jax.ShapeDtypeStruct((SEQ, HID), jnp.float32),
            jax.ShapeDtypeStruct((SEQ, E), jnp.int32),
            jax.ShapeDtypeStruct((SEQ, E), jnp.float32),
        ],
    )(ao, hs2d, pw, ln2w, ln2b, rw)


BR = 256


def _k3r_body(eidx_ref, pos0_ref, pos1_ref, te_ref):
    x = eidx_ref[...]
    laneE = lax.broadcasted_iota(jnp.int32, (SEQ, E), 1)
    m0 = (x[:, 0:1] == laneE).astype(jnp.float32)
    m1 = (x[:, 1:2] == laneE).astype(jnp.float32)
    m = jnp.concatenate([m0, m1], axis=1)

    ri = lax.broadcasted_iota(jnp.int32, (BR, BR), 0)
    ci = lax.broadcasted_iota(jnp.int32, (BR, BR), 1)
    tri = (ci < ri).astype(jnp.float32)

    carry = jnp.zeros((1, 2 * E), jnp.float32)
    blocks = []
    for b in range(SEQ // BR):
        mb = m[b * BR:(b + 1) * BR]
        blocks.append(jnp.dot(tri, mb, preferred_element_type=jnp.float32)
                      + carry)
        carry = carry + jnp.sum(mb, axis=0, keepdims=True)
    s = jnp.concatenate(blocks, axis=0)

    ct0 = carry[:, :E]
    ctot = carry[:, :E] + carry[:, E:]
    psi = ((ctot.astype(jnp.int32) + (ET - 1)) // ET) * ET
    ps = psi.astype(jnp.float32)
    i8 = lax.broadcasted_iota(jnp.int32, (E, E), 0)
    j8 = lax.broadcasted_iota(jnp.int32, (E, E), 1)
    tri8 = (i8 < j8).astype(jnp.float32)
    off = jnp.dot(ps, tri8, preferred_element_type=jnp.float32)

    pos0 = jnp.sum(m0 * (off + s[:, :E]), axis=1, keepdims=True)
    pos1 = jnp.sum(m1 * (off + ct0 + s[:, E:]), axis=1, keepdims=True)
    pos0_ref[...] = pos0.astype(jnp.int32)
    pos1_ref[...] = pos1.astype(jnp.int32)

    tv = (lax.broadcasted_iota(jnp.int32, (NTE, E), 0) * ET).astype(jnp.float32)
    cnt = jnp.sum((off <= tv).astype(jnp.int32), axis=1, keepdims=True) - 1
    nvalid = jnp.sum(psi, axis=1, keepdims=True) // ET
    idxn = lax.broadcasted_iota(jnp.int32, (NTE, 1), 0)
    te_ref[...] = jnp.where(idxn == NT, nvalid, jnp.clip(cnt, 0, E - 1))


def _k3r(eidx):
    return pl.pallas_call(
        _k3r_body,
        in_specs=[pl.BlockSpec((SEQ, E), lambda: (0, 0))],
        out_specs=[
            pl.BlockSpec((SEQ, 1), lambda: (0, 0)),
            pl.BlockSpec((SEQ, 1), lambda: (0, 0)),
            pl.BlockSpec((NTE, 1), lambda: (0, 0)),
        ],
        out_shape=[
            jax.ShapeDtypeStruct((SEQ, 1), jnp.int32),
            jax.ShapeDtypeStruct((SEQ, 1), jnp.int32),
            jax.ShapeDtypeStruct((NTE, 1), jnp.int32),
        ],
    )(eidx)


def _sca_body(pos0_hbm, pos1_hbm, flat_hbm, xp_hbm,
              posb0, posb1, rowbuf, sem):
    wid = lax.axis_index("s") * NC + lax.axis_index("c")
    tok_base = wid * TPW
    pltpu.sync_copy(pos0_hbm.at[pl.ds(tok_base, TPW)], posb0)
    pltpu.sync_copy(pos1_hbm.at[pl.ds(tok_base, TPW)], posb1)
    for g in range(TPW // L):
        pltpu.sync_copy(flat_hbm.at[pl.ds(tok_base + g * L, L)], rowbuf)
        p0 = posb0[pl.ds(g * L, L)]
        pltpu.async_copy(rowbuf, xp_hbm.at[p0], sem).wait()
        p1 = posb1[pl.ds(g * L, L)]
        pltpu.async_copy(rowbuf, xp_hbm.at[p1], sem).wait()


def _sc_scatter(pos0, pos1, flat):
    mesh = plsc.VectorSubcoreMesh(core_axis_name="c", subcore_axis_name="s",
                                  num_cores=NC, num_subcores=NS)
    f = pl.kernel(
        _sca_body,
        out_type=jax.ShapeDtypeStruct((P, HID), jnp.float32),
        mesh=mesh,
        scratch_types=[
            pltpu.VMEM((TPW,), jnp.int32),
            pltpu.VMEM((TPW,), jnp.int32),
            pltpu.VMEM((L, HID), jnp.float32),
            pltpu.SemaphoreType.DMA,
        ],
    )
    return f(pos0, pos1, flat)


def _k4_body(te_ref, x_ref, w1_ref, w2_ref, y_ref, w1b, w2b):
    i = pl.program_id(0)

    @pl.when(i < te_ref[NT])
    def _():
        prev = te_ref[jnp.maximum(i - 1, 0)]

        @pl.when((i == 0) | (te_ref[i] != prev))
        def _():
            w1b[...] = w1_ref[0].astype(jnp.bfloat16)
            w2b[...] = w2_ref[0].astype(jnp.bfloat16)

        xb = x_ref[...].astype(jnp.bfloat16)
        h = jnp.dot(xb, w1b[...], preferred_element_type=jnp.float32)
        h = jax.nn.gelu(h)
        y_ref[...] = jnp.dot(h.astype(jnp.bfloat16), w2b[...],
                             preferred_element_type=jnp.float32)


def _k4(te, xp, w1, w2):
    grid_spec = pltpu.PrefetchScalarGridSpec(
        num_scalar_prefetch=1,
        grid=(NT,),
        in_specs=[
            pl.BlockSpec((ET, HID), lambda i, te: (i, 0)),
            pl.BlockSpec((1, HID, DFF), lambda i, te: (te[i], 0, 0)),
            pl.BlockSpec((1, DFF, HID), lambda i, te: (te[i], 0, 0)),
        ],
        out_specs=pl.BlockSpec((ET, HID), lambda i, te: (i, 0)),
        scratch_shapes=[
            pltpu.VMEM((HID, DFF), jnp.bfloat16),
            pltpu.VMEM((DFF, HID), jnp.bfloat16),
        ],
    )
    return pl.pallas_call(
        _k4_body,
        grid_spec=grid_spec,
        out_shape=jax.ShapeDtypeStruct((P, HID), jnp.float32),
    )(te, xp, w1, w2)


def _scb_body(yp_hbm, pos0_hbm, pos1_hbm, y0_hbm, y1_hbm,
              idxv, rows, sem):
    wid = lax.axis_index("s") * NC + lax.axis_index("c")
    base = wid * TPW
    pltpu.sync_copy(pos0_hbm.at[pl.ds(base, TPW)], idxv)
    pltpu.async_copy(yp_hbm.at[idxv], rows, sem).wait()
    pltpu.sync_copy(rows, y0_hbm.at[pl.ds(base, TPW)])
    pltpu.sync_copy(pos1_hbm.at[pl.ds(base, TPW)], idxv)
    pltpu.async_copy(yp_hbm.at[idxv], rows, sem).wait()
    pltpu.sync_copy(rows, y1_hbm.at[pl.ds(base, TPW)])


def _sc_gather(yp, pos0, pos1):
    mesh = plsc.VectorSubcoreMesh(core_axis_name="c", subcore_axis_name="s", num_cores=NC, num_subcores=NS)
    f = pl.kernel(
        _scb_body,
        out_type=[
            jax.ShapeDtypeStruct((SEQ, HID), jnp.float32),
            jax.ShapeDtypeStruct((SEQ, HID), jnp.float32),
        ],
        mesh=mesh,
        scratch_types=[
            pltpu.VMEM((TPW,), jnp.int32),
            pltpu.VMEM((TPW, HID), jnp.float32),
            pltpu.SemaphoreType.DMA,
        ],
    )
    return f(yp, pos0, pos1)


def _k5_body(ha_ref, y0_ref, y1_ref, w0_ref, w1_ref, o_ref):
    o_ref[...] = (ha_ref[...] + w0_ref[...] * y0_ref[...]
                  + w1_ref[...] * y1_ref[...])


def _k5(ha, y0, y1, w0, w1):
    return pl.pallas_call(
        _k5_body,
        grid=(SEQ // ST,),
        in_specs=[
            pl.BlockSpec((ST, HID), lambda i: (i, 0)),
            pl.BlockSpec((ST, HID), lambda i: (i, 0)),
            pl.BlockSpec((ST, HID), lambda i: (i, 0)),
            pl.BlockSpec((ST, 1), lambda i: (i, 0)),
            pl.BlockSpec((ST, 1), lambda i: (i, 0)),
        ],
        out_specs=pl.BlockSpec((ST, HID), lambda i: (i, 0)),
        out_shape=jax.ShapeDtypeStruct((SEQ, HID), jnp.float32),
    )(ha, y0, y1, w0, w1)


def kernel(hidden_states, ln1_weight, ln1_bias, ln2_weight, ln2_bias,
           qkv_weight, proj_weight, router_weight, moe_w1, moe_w2):
    hs2d = hidden_states.reshape(SEQ, HID)
    w3 = qkv_weight.reshape(NH, 3, HD, HID)
    wq = w3[:, 0].reshape(NH * HD, HID).T
    wk = w3[:, 1].reshape(NH * HD, HID).T
    wv = w3[:, 2].reshape(NH * HD, HID).T
    q3, k3, v3 = _k1(hs2d, ln1_weight.reshape(1, HID),
                     ln1_bias.reshape(1, HID), wq, wk, wv)
    ao = _k2(q3, k3, v3)
    ha, flat, eidx, wgt = _k3(ao, hs2d, proj_weight,
                              ln2_weight.reshape(1, HID),
                              ln2_bias.reshape(1, HID), router_weight)
    pos0_2d, pos1_2d, te_2d = _k3r(eidx)
    pos0 = pos0_2d.reshape(SEQ)
    pos1 = pos1_2d.reshape(SEQ)
    te = te_2d.reshape(NTE)
    xp = _sc_scatter(pos0, pos1, flat)
    yp = _k4(te, xp, moe_w1, moe_w2)
    y0, y1 = _sc_gather(yp, pos0, pos1)
    out = _k5(ha, y0, y1, wgt[:, 0:1], wgt[:, 1:2])
    return out.reshape(SEQ, 1, HID)
